# Initial kernel scaffold; baseline (speedup 1.0000x reference)
import jax
import jax.numpy as jnp
from jax.experimental import pallas as pl


def kernel(x, edge_index, batch, W1, b1, W2, b2, Wf1, bf1, Wf2, bf2, Wo, bo):
    raise NotImplementedError("write your pallas kernel here")


# speedup vs baseline: 3.1718x; 3.1718x over previous
import functools

import jax
import jax.numpy as jnp
from jax import lax
from jax.experimental import pallas as pl
from jax.experimental.pallas import tpu as pltpu
from jax.experimental.pallas import tpu_sc as plsc

N = 10000
E = 160000
D = 256
G = 64

NSUB = 16
K = 128
EPS = 10112
EPAD = NSUB * EPS
NCHUNK = EPS // K
ACC_ROWS = 10240
ZROWS = 640

BR = 2000
NB = N // BR


def _mm_split_body(x_ref, w_ref, o_ref):
    m = jnp.dot(x_ref[...], w_ref[...], preferred_element_type=jnp.float32)
    o_ref[0] = m[:, :128]
    o_ref[1] = m[:, 128:]


def _mm_split(x, w):
    return pl.pallas_call(
        _mm_split_body,
        grid=(NB,),
        in_specs=[
            pl.BlockSpec((BR, D), lambda i: (i, 0)),
            pl.BlockSpec((D, D), lambda i: (0, 0)),
        ],
        out_specs=pl.BlockSpec((2, BR, 128), lambda i: (0, i, 0)),
        out_shape=jax.ShapeDtypeStruct((2, N, 128), jnp.float32),
    )(x, w)


def _norm_mm_body(agg_ref, b_ref, w_ref, o_ref):
    a = jnp.concatenate([agg_ref[0], agg_ref[1]], axis=1)
    h = jnp.maximum(a + b_ref[...], 0.0)
    n = jnp.sqrt(jnp.sum(h * h, axis=1, keepdims=True))
    h = h / jnp.maximum(n, 1e-12)
    m = jnp.dot(h, w_ref[...], preferred_element_type=jnp.float32)
    o_ref[0] = m[:, :128]
    o_ref[1] = m[:, 128:]


def _norm_mm(agg, b, w):
    return pl.pallas_call(
        _norm_mm_body,
        grid=(NB,),
        in_specs=[
            pl.BlockSpec((2, BR, 128), lambda i: (0, i, 0)),
            pl.BlockSpec((1, D), lambda i: (0, 0)),
            pl.BlockSpec((D, D), lambda i: (0, 0)),
        ],
        out_specs=pl.BlockSpec((2, BR, 128), lambda i: (0, i, 0)),
        out_shape=jax.ShapeDtypeStruct((2, N, 128), jnp.float32),
    )(agg, b.reshape(1, D), w)


def _pool_mlp_body(agg_ref, b_ref, batch_ref, wf1_ref, bf1_ref, wf2_ref,
                   bf2_ref, wo_ref, bo_ref, o_ref, sums_ref, cnt_ref):
    i = pl.program_id(0)

    a = jnp.concatenate([agg_ref[0], agg_ref[1]], axis=1)
    h = jnp.maximum(a + b_ref[...], 0.0)
    n = jnp.sqrt(jnp.sum(h * h, axis=1, keepdims=True))
    h = h / jnp.maximum(n, 1e-12)

    bb = batch_ref[0, 0]
    oh = (bb[:, None] == lax.broadcasted_iota(jnp.int32, (BR, G), 1)
          ).astype(jnp.float32)

    @pl.when(i == 0)
    def _():
        sums_ref[...] = jnp.zeros_like(sums_ref)
        cnt_ref[...] = jnp.zeros_like(cnt_ref)

    dn = (((0,), (0,)), ((), ()))
    sums_ref[...] += lax.dot_general(oh, h, dn,
                                     preferred_element_type=jnp.float32)
    cnt_ref[...] += lax.dot_general(oh, jnp.ones((BR, 128), jnp.float32), dn,
                                    preferred_element_type=jnp.float32)

    @pl.when(i == NB - 1)
    def _():
        mean = sums_ref[...] / jnp.maximum(cnt_ref[...][:, :1], 1.0)
        f = jnp.maximum(
            jnp.dot(mean, wf1_ref[...], preferred_element_type=jnp.float32)
            + bf1_ref[...], 0.0)
        f = jnp.maximum(
            jnp.dot(f, wf2_ref[...], preferred_element_type=jnp.float32)
            + bf2_ref[...], 0.0)
        o_ref[...] = (jnp.dot(f, wo_ref[...],
                              preferred_element_type=jnp.float32)
                      + bo_ref[...])


def _pool_mlp(agg, b, batch, wf1, bf1, wf2, bf2, wo, bo):
    return pl.pallas_call(
        _pool_mlp_body,
        grid=(NB,),
        in_specs=[
            pl.BlockSpec((2, BR, 128), lambda i: (0, i, 0)),
            pl.BlockSpec((1, D), lambda i: (0, 0)),
            pl.BlockSpec((1, 1, BR), lambda i: (i, 0, 0)),
            pl.BlockSpec((D, D), lambda i: (0, 0)),
            pl.BlockSpec((1, D), lambda i: (0, 0)),
            pl.BlockSpec((D, 128), lambda i: (0, 0)),
            pl.BlockSpec((1, 128), lambda i: (0, 0)),
            pl.BlockSpec((128, 1), lambda i: (0, 0)),
            pl.BlockSpec((1, 1), lambda i: (0, 0)),
        ],
        out_specs=pl.BlockSpec((G, 1), lambda i: (0, 0)),
        out_shape=jax.ShapeDtypeStruct((G, 1), jnp.float32),
        scratch_shapes=[
            pltpu.VMEM((G, D), jnp.float32),
            pltpu.VMEM((G, 128), jnp.float32),
        ],
    )(agg, b.reshape(1, D), batch.reshape(NB, 1, BR), wf1,
      bf1.reshape(1, D), wf2, bf2.reshape(1, 128), wo.reshape(128, 1),
      bo.reshape(1, 1))


def _seg_body(mtab_hbm, src_hbm, dst_hbm, out_hbm, sbuf, dbuf, rows, zbuf,
              acc):
    c = lax.axis_index("c")
    s = lax.axis_index("s")

    @pl.loop(0, 16)
    def _(r):
        for j in range(8):
            zbuf[r, pl.ds(j * 16, 16)] = jnp.zeros((16,), jnp.float32)

    @pl.loop(0, ZROWS // 16)
    def _(t):
        pltpu.sync_copy(zbuf, acc.at[pl.ds(s * ZROWS + t * 16, 16)])

    plsc.subcore_barrier()

    @pl.loop(0, NCHUNK)
    def _(i):
        base = s * EPS + i * K
        pltpu.sync_copy(src_hbm.at[c, pl.ds(base, K)], sbuf)
        pltpu.sync_copy(dst_hbm.at[pl.ds(base, K)], dbuf)
        pltpu.sync_copy(mtab_hbm.at[sbuf], rows)
        pltpu.sync_copy(rows, acc.at[dbuf], add=True)

    plsc.subcore_barrier()

    @pl.when(s < NSUB - 1)
    def _():
        pltpu.sync_copy(acc.at[pl.ds(s * ZROWS, ZROWS)],
                        out_hbm.at[c, pl.ds(s * ZROWS, ZROWS)])

    @pl.when(s == NSUB - 1)
    def _():
        r0 = (NSUB - 1) * ZROWS
        pltpu.sync_copy(acc.at[pl.ds(r0, N - r0)],
                        out_hbm.at[c, pl.ds(r0, N - r0)])


def _segment_sum_sc(mtab, src2, dstp):
    mesh = plsc.VectorSubcoreMesh(core_axis_name="c", subcore_axis_name="s")
    k = pl.kernel(
        _seg_body,
        out_type=jax.ShapeDtypeStruct((2, N, 128), jnp.float32),
        mesh=mesh,
        scratch_types=[
            pltpu.VMEM((K,), jnp.int32),
            pltpu.VMEM((K,), jnp.int32),
            pltpu.VMEM((K, 128), jnp.float32),
            pltpu.VMEM((16, 128), jnp.float32),
            pltpu.VMEM_SHARED((ACC_ROWS, 128), jnp.float32),
        ],
    )
    return k(mtab, src2, dstp)


def kernel(x, edge_index, batch, W1, b1, W2, b2, Wf1, bf1, Wf2, bf2, Wo, bo):
    src = edge_index[0].astype(jnp.int32)
    dst = edge_index[1].astype(jnp.int32)
    pad = EPAD - E
    src_p = jnp.concatenate([src, jnp.zeros((pad,), jnp.int32)])
    src2 = jnp.stack([src_p, src_p + N])
    dstp = jnp.concatenate([dst, jnp.full((pad,), N, jnp.int32)])

    m1 = _mm_split(x, W1)
    agg1 = _segment_sum_sc(m1.reshape(2 * N, 128), src2, dstp)
    m2 = _norm_mm(agg1, b1, W2)
    agg2 = _segment_sum_sc(m2.reshape(2 * N, 128), src2, dstp)
    return _pool_mlp(agg2, b2, batch.astype(jnp.int32), Wf1, bf1, Wf2, bf2,
                     Wo, bo)


# speedup vs baseline: 3.4558x; 1.0895x over previous
import functools

import jax
import jax.numpy as jnp
from jax import lax
from jax.experimental import pallas as pl
from jax.experimental.pallas import tpu as pltpu
from jax.experimental.pallas import tpu_sc as plsc

N = 10000
E = 160000
D = 256
G = 64

NSUB = 16
K = 128
EPS = 10240
EPAD = NSUB * EPS
NCHUNK = EPS // K
ACC_ROWS = 10240
ZROWS = 640

BR = 2000
NB = N // BR


def _mm_split_body(x_ref, w_ref, o_ref):
    m = jnp.dot(x_ref[...], w_ref[...], preferred_element_type=jnp.float32)
    o_ref[0] = m[:, :128]
    o_ref[1] = m[:, 128:]


def _mm_split(x, w):
    return pl.pallas_call(
        _mm_split_body,
        grid=(NB,),
        in_specs=[
            pl.BlockSpec((BR, D), lambda i: (i, 0)),
            pl.BlockSpec((D, D), lambda i: (0, 0)),
        ],
        out_specs=pl.BlockSpec((2, BR, 128), lambda i: (0, i, 0)),
        out_shape=jax.ShapeDtypeStruct((2, N, 128), jnp.float32),
    )(x, w)


def _norm_mm_body(agg_ref, b_ref, w_ref, o_ref):
    a = jnp.concatenate([agg_ref[0], agg_ref[1]], axis=1)
    h = jnp.maximum(a + b_ref[...], 0.0)
    n = jnp.sqrt(jnp.sum(h * h, axis=1, keepdims=True))
    h = h / jnp.maximum(n, 1e-12)
    m = jnp.dot(h, w_ref[...], preferred_element_type=jnp.float32)
    o_ref[0] = m[:, :128]
    o_ref[1] = m[:, 128:]


def _norm_mm(agg, b, w):
    return pl.pallas_call(
        _norm_mm_body,
        grid=(NB,),
        in_specs=[
            pl.BlockSpec((2, BR, 128), lambda i: (0, i, 0)),
            pl.BlockSpec((1, D), lambda i: (0, 0)),
            pl.BlockSpec((D, D), lambda i: (0, 0)),
        ],
        out_specs=pl.BlockSpec((2, BR, 128), lambda i: (0, i, 0)),
        out_shape=jax.ShapeDtypeStruct((2, N, 128), jnp.float32),
    )(agg, b.reshape(1, D), w)


def _pool_mlp_body(agg_ref, b_ref, batch_ref, wf1_ref, bf1_ref, wf2_ref,
                   bf2_ref, wo_ref, bo_ref, o_ref, sums_ref, cnt_ref):
    i = pl.program_id(0)

    a = jnp.concatenate([agg_ref[0], agg_ref[1]], axis=1)
    h = jnp.maximum(a + b_ref[...], 0.0)
    n = jnp.sqrt(jnp.sum(h * h, axis=1, keepdims=True))
    h = h / jnp.maximum(n, 1e-12)

    bb = batch_ref[0, 0]
    oh = (bb[:, None] == lax.broadcasted_iota(jnp.int32, (BR, G), 1)
          ).astype(jnp.float32)

    @pl.when(i == 0)
    def _():
        sums_ref[...] = jnp.zeros_like(sums_ref)
        cnt_ref[...] = jnp.zeros_like(cnt_ref)

    dn = (((0,), (0,)), ((), ()))
    sums_ref[...] += lax.dot_general(oh, h, dn,
                                     preferred_element_type=jnp.float32)
    cnt_ref[...] += lax.dot_general(oh, jnp.ones((BR, 128), jnp.float32), dn,
                                    preferred_element_type=jnp.float32)

    @pl.when(i == NB - 1)
    def _():
        mean = sums_ref[...] / jnp.maximum(cnt_ref[...][:, :1], 1.0)
        f = jnp.maximum(
            jnp.dot(mean, wf1_ref[...], preferred_element_type=jnp.float32)
            + bf1_ref[...], 0.0)
        f = jnp.maximum(
            jnp.dot(f, wf2_ref[...], preferred_element_type=jnp.float32)
            + bf2_ref[...], 0.0)
        o_ref[...] = (jnp.dot(f, wo_ref[...],
                              preferred_element_type=jnp.float32)
                      + bo_ref[...])


def _pool_mlp(agg, b, batch, wf1, bf1, wf2, bf2, wo, bo):
    return pl.pallas_call(
        _pool_mlp_body,
        grid=(NB,),
        in_specs=[
            pl.BlockSpec((2, BR, 128), lambda i: (0, i, 0)),
            pl.BlockSpec((1, D), lambda i: (0, 0)),
            pl.BlockSpec((1, 1, BR), lambda i: (i, 0, 0)),
            pl.BlockSpec((D, D), lambda i: (0, 0)),
            pl.BlockSpec((1, D), lambda i: (0, 0)),
            pl.BlockSpec((D, 128), lambda i: (0, 0)),
            pl.BlockSpec((1, 128), lambda i: (0, 0)),
            pl.BlockSpec((128, 1), lambda i: (0, 0)),
            pl.BlockSpec((1, 1), lambda i: (0, 0)),
        ],
        out_specs=pl.BlockSpec((G, 1), lambda i: (0, 0)),
        out_shape=jax.ShapeDtypeStruct((G, 1), jnp.float32),
        scratch_shapes=[
            pltpu.VMEM((G, D), jnp.float32),
            pltpu.VMEM((G, 128), jnp.float32),
        ],
    )(agg, b.reshape(1, D), batch.reshape(NB, 1, BR), wf1,
      bf1.reshape(1, D), wf2, bf2.reshape(1, 128), wo.reshape(128, 1),
      bo.reshape(1, 1))


def _seg_body(mtab_hbm, sd_hbm, out_hbm, ibuf, rows2, zbuf, acc, isem0,
              isem1, gsem0, gsem1):
    c = lax.axis_index("c")
    s = lax.axis_index("s")
    isems = (isem0, isem1)
    gsems = (gsem0, gsem1)

    @pl.loop(0, 16)
    def _(r):
        for j in range(8):
            zbuf[r, pl.ds(j * 16, 16)] = jnp.zeros((16,), jnp.float32)

    @pl.loop(0, ZROWS // 16)
    def _(t):
        pltpu.sync_copy(zbuf, acc.at[pl.ds(s * ZROWS + t * 16, 16)])

    pltpu.sync_copy(sd_hbm.at[c, s, 0], ibuf.at[0])
    pltpu.async_copy(sd_hbm.at[c, s, 1], ibuf.at[1], isem1)
    pltpu.async_copy(mtab_hbm.at[ibuf.at[0, 0]], rows2.at[0], gsem0)

    plsc.subcore_barrier()

    @pl.loop(0, NCHUNK, step=2)
    def _(i):
        for b in range(2):
            j = i + b
            nb = 1 - b

            pltpu.make_async_copy(mtab_hbm.at[ibuf.at[b, 0]], rows2.at[b],
                                  gsems[b]).wait()

            @pl.when(j < NCHUNK - 1)
            def _():
                pltpu.make_async_copy(sd_hbm.at[c, s, j + 1], ibuf.at[nb],
                                      isems[nb]).wait()
                pltpu.async_copy(mtab_hbm.at[ibuf.at[nb, 0]], rows2.at[nb],
                                 gsems[nb])

            pltpu.sync_copy(rows2.at[b], acc.at[ibuf.at[b, 1]], add=True)

            @pl.when(j < NCHUNK - 2)
            def _():
                pltpu.async_copy(sd_hbm.at[c, s, j + 2], ibuf.at[b],
                                 isems[b])

    plsc.subcore_barrier()

    @pl.when(s < NSUB - 1)
    def _():
        pltpu.sync_copy(acc.at[pl.ds(s * ZROWS, ZROWS)],
                        out_hbm.at[c, pl.ds(s * ZROWS, ZROWS)])

    @pl.when(s == NSUB - 1)
    def _():
        r0 = (NSUB - 1) * ZROWS
        pltpu.sync_copy(acc.at[pl.ds(r0, N - r0)],
                        out_hbm.at[c, pl.ds(r0, N - r0)])


def _segment_sum_sc(mtab, sd):
    mesh = plsc.VectorSubcoreMesh(core_axis_name="c", subcore_axis_name="s")
    k = pl.kernel(
        _seg_body,
        out_type=jax.ShapeDtypeStruct((2, N, 128), jnp.float32),
        mesh=mesh,
        scratch_types=[
            pltpu.VMEM((2, 2, K), jnp.int32),
            pltpu.VMEM((2, K, 128), jnp.float32),
            pltpu.VMEM((16, 128), jnp.float32),
            pltpu.VMEM_SHARED((ACC_ROWS, 128), jnp.float32),
            pltpu.SemaphoreType.DMA,
            pltpu.SemaphoreType.DMA,
            pltpu.SemaphoreType.DMA,
            pltpu.SemaphoreType.DMA,
        ],
    )
    return k(mtab, sd)


def kernel(x, edge_index, batch, W1, b1, W2, b2, Wf1, bf1, Wf2, bf2, Wo, bo):
    src = edge_index[0].astype(jnp.int32)
    dst = edge_index[1].astype(jnp.int32)
    pad = EPAD - E
    src_p = jnp.concatenate([src, jnp.zeros((pad,), jnp.int32)])
    src2 = jnp.stack([src_p, src_p + N])
    dstp = jnp.concatenate([dst, jnp.full((pad,), N, jnp.int32)])
    s4 = src2.reshape(2, NSUB, NCHUNK, 1, K)
    d4 = jnp.broadcast_to(dstp.reshape(1, NSUB, NCHUNK, 1, K),
                          (2, NSUB, NCHUNK, 1, K))
    sd = jnp.concatenate([s4, d4], axis=3)

    m1 = _mm_split(x, W1)
    agg1 = _segment_sum_sc(m1.reshape(2 * N, 128), sd)
    m2 = _norm_mm(agg1, b1, W2)
    agg2 = _segment_sum_sc(m2.reshape(2 * N, 128), sd)
    return _pool_mlp(agg2, b2, batch.astype(jnp.int32), Wf1, bf1, Wf2, bf2,
                     Wo, bo)
